# baseline (speedup 1.0000x reference)
import functools

import jax
import jax.numpy as jnp
from jax.experimental import pallas as pl
from jax.experimental.pallas import tpu as pltpu


def _dense_body(n_experts, nf, x_ref, Wsw_ref, bsw_ref, W1_ref, b1_ref,
                W2_ref, b2_ref, out_ref, counts_ref, psum_ref, xs_ref,
                routes_ref, yacc_ref):
    e = pl.program_id(1)
    f = pl.program_id(2)

    @pl.when((e == 0) & (f == 0))
    def _router():
        x = x_ref[...]
        logits = jnp.dot(x, Wsw_ref[...],
                         preferred_element_type=jnp.float32) + bsw_ref[...]
        m = jnp.max(logits, axis=-1, keepdims=True)
        ex = jnp.exp(logits - m)
        p = ex / jnp.sum(ex, axis=-1, keepdims=True)
        pmax = jnp.max(p, axis=-1, keepdims=True)
        iota = jax.lax.broadcasted_iota(jnp.int32, p.shape, 1)
        routes = jnp.min(jnp.where(p >= pmax, iota, n_experts), axis=-1,
                         keepdims=True)
        routes_ref[...] = routes
        xs_ref[...] = x * pmax
        onehot = (routes == iota).astype(jnp.float32)

        @pl.when(pl.program_id(0) == 0)
        def _init():
            counts_ref[...] = jnp.zeros_like(counts_ref)
            psum_ref[...] = jnp.zeros_like(psum_ref)

        counts_ref[...] += jnp.sum(onehot, axis=0, keepdims=True)
        psum_ref[...] += jnp.sum(p, axis=0, keepdims=True)

    xs = xs_ref[...]
    h = jnp.maximum(
        jnp.dot(xs, W1_ref[0], preferred_element_type=jnp.float32)
        + b1_ref[0], 0.0)
    part = jnp.dot(h, W2_ref[0], preferred_element_type=jnp.float32)

    @pl.when(f == 0)
    def _init_acc():
        yacc_ref[...] = b2_ref[0] + jnp.zeros_like(yacc_ref)

    yacc_ref[...] += part

    @pl.when(f == nf - 1)
    def _write():
        sel = routes_ref[...] == e
        prev = jnp.where(e == 0, jnp.zeros_like(yacc_ref), out_ref[...])
        out_ref[...] = jnp.where(sel, yacc_ref[...], prev)


def kernel(x, W_switch, b_switch, W1, b1, W2, b2):
    seq_len, batch, d_model = x.shape
    n_experts, _, d_ff = W1.shape
    n = seq_len * batch
    tb = 512 if n % 512 == 0 else n
    ntb = n // tb
    fb = 1024 if d_ff % 1024 == 0 else d_ff
    nf = d_ff // fb
    xf = x.reshape(n, d_model)

    out, counts, psum = pl.pallas_call(
        functools.partial(_dense_body, n_experts, nf),
        grid=(ntb, n_experts, nf),
        in_specs=[
            pl.BlockSpec((tb, d_model), lambda t, e, f: (t, 0)),
            pl.BlockSpec((d_model, n_experts), lambda t, e, f: (0, 0)),
            pl.BlockSpec((1, n_experts), lambda t, e, f: (0, 0)),
            pl.BlockSpec((1, d_model, fb), lambda t, e, f: (e, 0, f)),
            pl.BlockSpec((1, 1, fb), lambda t, e, f: (e, 0, f)),
            pl.BlockSpec((1, fb, d_model), lambda t, e, f: (e, f, 0)),
            pl.BlockSpec((1, 1, d_model), lambda t, e, f: (e, 0, 0)),
        ],
        out_specs=[
            pl.BlockSpec((tb, d_model), lambda t, e, f: (t, 0)),
            pl.BlockSpec((1, n_experts), lambda t, e, f: (0, 0)),
            pl.BlockSpec((1, n_experts), lambda t, e, f: (0, 0)),
        ],
        out_shape=[
            jax.ShapeDtypeStruct((n, d_model), jnp.float32),
            jax.ShapeDtypeStruct((1, n_experts), jnp.float32),
            jax.ShapeDtypeStruct((1, n_experts), jnp.float32),
        ],
        scratch_shapes=[
            pltpu.VMEM((tb, d_model), jnp.float32),
            pltpu.VMEM((tb, 1), jnp.int32),
            pltpu.VMEM((tb, d_model), jnp.float32),
        ],
    )(xf, W_switch, b_switch.reshape(1, n_experts),
      W1, b1.reshape(n_experts, 1, d_ff), W2, b2.reshape(n_experts, 1, d_model))

    return (out.reshape(x.shape), counts.reshape(n_experts),
            psum.reshape(n_experts), jnp.array(0, dtype=jnp.int32))


# speedup vs baseline: 1.9583x; 1.9583x over previous
import functools

import jax
import jax.numpy as jnp
from jax import lax
from jax.experimental import pallas as pl
from jax.experimental.pallas import tpu as pltpu
from jax.experimental.pallas import tpu_sc as plsc

_NC, _NS = 2, 16
_NW = _NC * _NS
_BLK = 128


def _router_body(n_experts, x_ref, Wsw_ref, bsw_ref, xs_ref, routes_ref,
                 rl_ref, hist_ref, psum_ref):
    x = x_ref[...]
    logits = jnp.dot(x, Wsw_ref[...],
                     preferred_element_type=jnp.float32) + bsw_ref[...]
    m = jnp.max(logits, axis=-1, keepdims=True)
    ex = jnp.exp(logits - m)
    p = ex / jnp.sum(ex, axis=-1, keepdims=True)
    pmax = jnp.max(p, axis=-1, keepdims=True)
    iota_e = lax.broadcasted_iota(jnp.int32, p.shape, 1)
    routes = jnp.min(jnp.where(p >= pmax, iota_e, n_experts), axis=-1,
                     keepdims=True)
    xs_ref[...] = (x * pmax).astype(jnp.bfloat16)
    routes_ref[...] = routes
    onehot = (routes == iota_e).astype(jnp.float32)
    tb = x.shape[0]
    ri = lax.broadcasted_iota(jnp.int32, (tb, tb), 0)
    ci = lax.broadcasted_iota(jnp.int32, (tb, tb), 1)
    tri = (ri > ci).astype(jnp.float32)
    before = jnp.dot(tri, onehot, preferred_element_type=jnp.float32)
    rl = jnp.sum(before * onehot, axis=-1, keepdims=True)
    rl_ref[...] = rl.astype(jnp.int32)
    hist_ref[...] = jnp.sum(onehot, axis=0).reshape(1, 1, n_experts)

    @pl.when(pl.program_id(0) == 0)
    def _():
        psum_ref[...] = jnp.zeros_like(psum_ref)

    psum_ref[...] += jnp.sum(p, axis=0, keepdims=True)


def _dispatch_body(xs_hbm, routes_hbm, rl_hbm, tflat_hbm, xsorted_hbm,
                   pos_hbm, routes_v, rl_v, t_v, pm0, pm1, pm2, pm3,
                   rows_v, sem):
    w = lax.axis_index("s") * _NC + lax.axis_index("c")
    base = w * 256
    pltpu.sync_copy(routes_hbm.at[pl.ds(base, 256)], routes_v)
    pltpu.sync_copy(rl_hbm.at[pl.ds(base, 256)], rl_v)
    pltpu.sync_copy(tflat_hbm.at[pl.ds(w * 8, 8)], t_v)
    pms = [pm0, pm1, pm2, pm3]
    for j in range(4):
        for g in range(4):
            o = j * 64 + g * 16
            rv = routes_v[pl.ds(o, 16)]
            bv = plsc.load_gather(t_v, [rv])
            pms[j][pl.ds(g * 16, 16)] = bv + rl_v[pl.ds(o, 16)]
    for j in range(4):
        pltpu.sync_copy(pms[j], pos_hbm.at[w * 4 + j])
        pltpu.sync_copy(xs_hbm.at[pl.ds(base + j * 64, 64)], rows_v)
        pltpu.async_copy(rows_v, xsorted_hbm.at[pms[j]], sem).wait()


def _ffn_body(be_ref, x_ref, W1_ref, b1_ref, W2_ref, b2_ref, out_ref):
    xb = x_ref[...]
    h = jnp.maximum(
        jnp.dot(xb, W1_ref[0], preferred_element_type=jnp.float32)
        + b1_ref[0], 0.0)
    y = jnp.dot(h.astype(jnp.bfloat16), W2_ref[0],
                preferred_element_type=jnp.float32) + b2_ref[0]
    out_ref[...] = y


def _combine_body(ysorted_hbm, pos_hbm, final_hbm, pm0, pm1, pm2, pm3,
                  rows_v, sem):
    w = lax.axis_index("s") * _NC + lax.axis_index("c")
    base = w * 256
    pms = [pm0, pm1, pm2, pm3]
    for j in range(4):
        pltpu.sync_copy(pos_hbm.at[w * 4 + j], pms[j])
        pltpu.async_copy(ysorted_hbm.at[pms[j]], rows_v, sem).wait()
        pltpu.sync_copy(rows_v, final_hbm.at[pl.ds(base + j * 64, 64)])


def kernel(x, W_switch, b_switch, W1, b1, W2, b2):
    seq_len, batch, d_model = x.shape
    n_experts, _, d_ff = W1.shape
    n = seq_len * batch
    tb = 1024
    ntb = n // tb
    xf = x.reshape(n, d_model)

    xs, routes2, rl2, hist3, psum = pl.pallas_call(
        functools.partial(_router_body, n_experts),
        grid=(ntb,),
        in_specs=[
            pl.BlockSpec((tb, d_model), lambda t: (t, 0)),
            pl.BlockSpec((d_model, n_experts), lambda t: (0, 0)),
            pl.BlockSpec((1, n_experts), lambda t: (0, 0)),
        ],
        out_specs=[
            pl.BlockSpec((tb, d_model), lambda t: (t, 0)),
            pl.BlockSpec((tb, 1), lambda t: (t, 0)),
            pl.BlockSpec((tb, 1), lambda t: (t, 0)),
            pl.BlockSpec((1, 1, n_experts), lambda t: (t, 0, 0)),
            pl.BlockSpec((1, n_experts), lambda t: (0, 0)),
        ],
        out_shape=[
            jax.ShapeDtypeStruct((n, d_model), jnp.bfloat16),
            jax.ShapeDtypeStruct((n, 1), jnp.int32),
            jax.ShapeDtypeStruct((n, 1), jnp.int32),
            jax.ShapeDtypeStruct((ntb, 1, n_experts), jnp.float32),
            jax.ShapeDtypeStruct((1, n_experts), jnp.float32),
        ],
    )(xf, W_switch, b_switch.reshape(1, n_experts))

    hist = hist3.reshape(ntb, n_experts)
    counts = jnp.sum(hist, axis=0)
    counts_i = counts.astype(jnp.int32)
    block_base = (jnp.cumsum(hist, axis=0) - hist).astype(jnp.int32)
    sizes = ((counts_i + (_BLK - 1)) // _BLK) * _BLK
    ends = jnp.cumsum(sizes)
    starts = ends - sizes
    p_tot = n + n_experts * _BLK
    nblk = p_tot // _BLK
    blk_lo = jnp.arange(nblk, dtype=jnp.int32) * _BLK
    block_expert = jnp.minimum(
        jnp.sum((blk_lo[:, None] >= ends[None, :]).astype(jnp.int32), axis=1),
        n_experts - 1).astype(jnp.int32)
    t_tab = starts[None, :] + jnp.repeat(block_base, tb // (n // _NW), axis=0)
    tflat = t_tab.reshape(-1).astype(jnp.int32)

    mesh = plsc.VectorSubcoreMesh(core_axis_name="c", subcore_axis_name="s")
    xs_i = lax.bitcast_convert_type(
        xs.reshape(n, d_model // 2, 2), jnp.int32)
    x_sorted_i, pos2d = pl.kernel(
        _dispatch_body,
        out_type=[
            jax.ShapeDtypeStruct((p_tot, d_model // 2), jnp.int32),
            jax.ShapeDtypeStruct((n // 64, 64), jnp.int32),
        ],
        mesh=mesh,
        compiler_params=pltpu.CompilerParams(needs_layout_passes=False),
        scratch_types=[
            pltpu.VMEM((256,), jnp.int32),
            pltpu.VMEM((256,), jnp.int32),
            pltpu.VMEM((8,), jnp.int32),
            pltpu.VMEM((64,), jnp.int32),
            pltpu.VMEM((64,), jnp.int32),
            pltpu.VMEM((64,), jnp.int32),
            pltpu.VMEM((64,), jnp.int32),
            pltpu.VMEM((64, d_model // 2), jnp.int32),
            pltpu.SemaphoreType.DMA,
        ],
    )(xs_i, routes2.reshape(n), rl2.reshape(n), tflat)
    x_sorted = lax.bitcast_convert_type(
        x_sorted_i, jnp.bfloat16).reshape(p_tot, d_model)

    grid_spec = pltpu.PrefetchScalarGridSpec(
        num_scalar_prefetch=1,
        grid=(nblk,),
        in_specs=[
            pl.BlockSpec((_BLK, d_model), lambda t, be: (t, 0)),
            pl.BlockSpec((1, d_model, d_ff), lambda t, be: (be[t], 0, 0)),
            pl.BlockSpec((1, 1, d_ff), lambda t, be: (be[t], 0, 0)),
            pl.BlockSpec((1, d_ff, d_model), lambda t, be: (be[t], 0, 0)),
            pl.BlockSpec((1, 1, d_model), lambda t, be: (be[t], 0, 0)),
        ],
        out_specs=pl.BlockSpec((_BLK, d_model), lambda t, be: (t, 0)),
    )
    y_sorted = pl.pallas_call(
        _ffn_body,
        grid_spec=grid_spec,
        out_shape=jax.ShapeDtypeStruct((p_tot, d_model), jnp.float32),
    )(block_expert, x_sorted, W1.astype(jnp.bfloat16),
      b1.reshape(n_experts, 1, d_ff), W2.astype(jnp.bfloat16),
      b2.reshape(n_experts, 1, d_model))

    final = pl.kernel(
        _combine_body,
        out_type=jax.ShapeDtypeStruct((n, d_model), jnp.float32),
        mesh=mesh,
        scratch_types=[
            pltpu.VMEM((64,), jnp.int32),
            pltpu.VMEM((64,), jnp.int32),
            pltpu.VMEM((64,), jnp.int32),
            pltpu.VMEM((64,), jnp.int32),
            pltpu.VMEM((64, d_model), jnp.float32),
            pltpu.SemaphoreType.DMA,
        ],
    )(y_sorted, pos2d)

    return (final.reshape(x.shape), counts, psum.reshape(n_experts),
            jnp.array(0, dtype=jnp.int32))


# speedup vs baseline: 3.1885x; 1.6282x over previous
import functools

import jax
import jax.numpy as jnp
from jax import lax
from jax.experimental import pallas as pl
from jax.experimental.pallas import tpu as pltpu
from jax.experimental.pallas import tpu_sc as plsc

_NC, _NS = 2, 16
_NW = _NC * _NS
_BLK = 128


def _router_body(n_experts, x_ref, Wsw_ref, bsw_ref, xs_ref, routes_ref,
                 rl_ref, hist_ref, psum_ref):
    x = x_ref[...]
    logits = jnp.dot(x, Wsw_ref[...],
                     preferred_element_type=jnp.float32) + bsw_ref[...]
    m = jnp.max(logits, axis=-1, keepdims=True)
    ex = jnp.exp(logits - m)
    p = ex / jnp.sum(ex, axis=-1, keepdims=True)
    pmax = jnp.max(p, axis=-1, keepdims=True)
    iota_e = lax.broadcasted_iota(jnp.int32, p.shape, 1)
    routes = jnp.min(jnp.where(p >= pmax, iota_e, n_experts), axis=-1,
                     keepdims=True)
    xs_ref[...] = x * pmax
    routes_ref[...] = routes
    onehot = (routes == iota_e).astype(jnp.float32)
    tb = x.shape[0]
    ri = lax.broadcasted_iota(jnp.int32, (tb, tb), 0)
    ci = lax.broadcasted_iota(jnp.int32, (tb, tb), 1)
    tri = (ri > ci).astype(jnp.float32)
    before = jnp.dot(tri, onehot, preferred_element_type=jnp.float32)
    rl = jnp.sum(before * onehot, axis=-1, keepdims=True)
    rl_ref[...] = rl.astype(jnp.int32)
    hist_ref[...] = jnp.sum(onehot, axis=0).reshape(1, 1, n_experts)

    @pl.when(pl.program_id(0) == 0)
    def _():
        psum_ref[...] = jnp.zeros_like(psum_ref)

    psum_ref[...] += jnp.sum(p, axis=0, keepdims=True)


def _dispatch_body(xs_hbm, routes_hbm, rl_hbm, tflat_hbm, xsorted_hbm,
                   pos_hbm, routes_v, rl_v, t_v, pm0, pm1, pm2, pm3,
                   rows_v, sem):
    w = lax.axis_index("s") * _NC + lax.axis_index("c")
    base = w * 256
    pltpu.sync_copy(routes_hbm.at[pl.ds(base, 256)], routes_v)
    pltpu.sync_copy(rl_hbm.at[pl.ds(base, 256)], rl_v)
    pltpu.sync_copy(tflat_hbm.at[pl.ds(w * 8, 8)], t_v)
    pms = [pm0, pm1, pm2, pm3]
    for j in range(4):
        for g in range(4):
            o = j * 64 + g * 16
            rv = routes_v[pl.ds(o, 16)]
            bv = plsc.load_gather(t_v, [rv])
            pms[j][pl.ds(g * 16, 16)] = bv + rl_v[pl.ds(o, 16)]
    for j in range(4):
        pltpu.sync_copy(pms[j], pos_hbm.at[w * 4 + j])
        pltpu.sync_copy(xs_hbm.at[pl.ds(base + j * 64, 64)], rows_v)
        pltpu.async_copy(rows_v, xsorted_hbm.at[pms[j]], sem).wait()


def _wconv_body(W1_ref, W2_ref, W1b_ref, W2b_ref):
    W1b_ref[...] = W1_ref[...].astype(jnp.bfloat16)
    W2b_ref[...] = W2_ref[...].astype(jnp.bfloat16)


def _ffn_body(be_ref, x_ref, W1_ref, b1_ref, W2_ref, b2_ref, out_ref):
    xb = x_ref[...].astype(jnp.bfloat16)
    h = jnp.maximum(
        jnp.dot(xb, W1_ref[0], preferred_element_type=jnp.float32)
        + b1_ref[0], 0.0)
    y = jnp.dot(h.astype(jnp.bfloat16), W2_ref[0],
                preferred_element_type=jnp.float32) + b2_ref[0]
    out_ref[...] = y


def _combine_body(ysorted_hbm, pos_hbm, final_hbm, pm0, pm1, pm2, pm3,
                  rows_v, sem):
    w = lax.axis_index("s") * _NC + lax.axis_index("c")
    base = w * 256
    pms = [pm0, pm1, pm2, pm3]
    for j in range(4):
        pltpu.sync_copy(pos_hbm.at[w * 4 + j], pms[j])
        pltpu.async_copy(ysorted_hbm.at[pms[j]], rows_v, sem).wait()
        pltpu.sync_copy(rows_v, final_hbm.at[pl.ds(base + j * 64, 64)])


def kernel(x, W_switch, b_switch, W1, b1, W2, b2):
    seq_len, batch, d_model = x.shape
    n_experts, _, d_ff = W1.shape
    n = seq_len * batch
    tb = 1024
    ntb = n // tb
    xf = x.reshape(n, d_model)

    xs, routes2, rl2, hist3, psum = pl.pallas_call(
        functools.partial(_router_body, n_experts),
        grid=(ntb,),
        in_specs=[
            pl.BlockSpec((tb, d_model), lambda t: (t, 0)),
            pl.BlockSpec((d_model, n_experts), lambda t: (0, 0)),
            pl.BlockSpec((1, n_experts), lambda t: (0, 0)),
        ],
        out_specs=[
            pl.BlockSpec((tb, d_model), lambda t: (t, 0)),
            pl.BlockSpec((tb, 1), lambda t: (t, 0)),
            pl.BlockSpec((tb, 1), lambda t: (t, 0)),
            pl.BlockSpec((1, 1, n_experts), lambda t: (t, 0, 0)),
            pl.BlockSpec((1, n_experts), lambda t: (0, 0)),
        ],
        out_shape=[
            jax.ShapeDtypeStruct((n, d_model), jnp.float32),
            jax.ShapeDtypeStruct((n, 1), jnp.int32),
            jax.ShapeDtypeStruct((n, 1), jnp.int32),
            jax.ShapeDtypeStruct((ntb, 1, n_experts), jnp.float32),
            jax.ShapeDtypeStruct((1, n_experts), jnp.float32),
        ],
    )(xf, W_switch, b_switch.reshape(1, n_experts))

    hist = hist3.reshape(ntb, n_experts)
    counts = jnp.sum(hist, axis=0)
    counts_i = counts.astype(jnp.int32)
    block_base = (jnp.cumsum(hist, axis=0) - hist).astype(jnp.int32)
    sizes = ((counts_i + (_BLK - 1)) // _BLK) * _BLK
    ends = jnp.cumsum(sizes)
    starts = ends - sizes
    p_tot = n + n_experts * _BLK
    nblk = p_tot // _BLK
    blk_lo = jnp.arange(nblk, dtype=jnp.int32) * _BLK
    block_expert = jnp.minimum(
        jnp.sum((blk_lo[:, None] >= ends[None, :]).astype(jnp.int32), axis=1),
        n_experts - 1).astype(jnp.int32)
    t_tab = starts[None, :] + jnp.repeat(block_base, tb // (n // _NW), axis=0)
    tflat = t_tab.reshape(-1).astype(jnp.int32)

    mesh = plsc.VectorSubcoreMesh(core_axis_name="c", subcore_axis_name="s")
    x_sorted, pos2d = pl.kernel(
        _dispatch_body,
        out_type=[
            jax.ShapeDtypeStruct((p_tot, d_model), jnp.float32),
            jax.ShapeDtypeStruct((n // 64, 64), jnp.int32),
        ],
        mesh=mesh,
        compiler_params=pltpu.CompilerParams(needs_layout_passes=False),
        scratch_types=[
            pltpu.VMEM((256,), jnp.int32),
            pltpu.VMEM((256,), jnp.int32),
            pltpu.VMEM((8,), jnp.int32),
            pltpu.VMEM((64,), jnp.int32),
            pltpu.VMEM((64,), jnp.int32),
            pltpu.VMEM((64,), jnp.int32),
            pltpu.VMEM((64,), jnp.int32),
            pltpu.VMEM((64, d_model), jnp.float32),
            pltpu.SemaphoreType.DMA,
        ],
    )(xs, routes2.reshape(n), rl2.reshape(n), tflat)

    nwc = 8
    W1b, W2b = pl.pallas_call(
        _wconv_body,
        grid=(n_experts, nwc),
        in_specs=[
            pl.BlockSpec((1, d_model, d_ff // nwc), lambda e, c: (e, 0, c)),
            pl.BlockSpec((1, d_ff // nwc, d_model), lambda e, c: (e, c, 0)),
        ],
        out_specs=[
            pl.BlockSpec((1, d_model, d_ff // nwc), lambda e, c: (e, 0, c)),
            pl.BlockSpec((1, d_ff // nwc, d_model), lambda e, c: (e, c, 0)),
        ],
        out_shape=[
            jax.ShapeDtypeStruct((n_experts, d_model, d_ff), jnp.bfloat16),
            jax.ShapeDtypeStruct((n_experts, d_ff, d_model), jnp.bfloat16),
        ],
    )(W1, W2)

    grid_spec = pltpu.PrefetchScalarGridSpec(
        num_scalar_prefetch=1,
        grid=(nblk,),
        in_specs=[
            pl.BlockSpec((_BLK, d_model), lambda t, be: (t, 0)),
            pl.BlockSpec((1, d_model, d_ff), lambda t, be: (be[t], 0, 0)),
            pl.BlockSpec((1, 1, d_ff), lambda t, be: (be[t], 0, 0)),
            pl.BlockSpec((1, d_ff, d_model), lambda t, be: (be[t], 0, 0)),
            pl.BlockSpec((1, 1, d_model), lambda t, be: (be[t], 0, 0)),
        ],
        out_specs=pl.BlockSpec((_BLK, d_model), lambda t, be: (t, 0)),
    )
    y_sorted = pl.pallas_call(
        _ffn_body,
        grid_spec=grid_spec,
        out_shape=jax.ShapeDtypeStruct((p_tot, d_model), jnp.float32),
    )(block_expert, x_sorted, W1b,
      b1.reshape(n_experts, 1, d_ff), W2b,
      b2.reshape(n_experts, 1, d_model))

    final = pl.kernel(
        _combine_body,
        out_type=jax.ShapeDtypeStruct((n, d_model), jnp.float32),
        mesh=mesh,
        scratch_types=[
            pltpu.VMEM((64,), jnp.int32),
            pltpu.VMEM((64,), jnp.int32),
            pltpu.VMEM((64,), jnp.int32),
            pltpu.VMEM((64,), jnp.int32),
            pltpu.VMEM((64, d_model), jnp.float32),
            pltpu.SemaphoreType.DMA,
        ],
    )(y_sorted, pos2d)

    return (final.reshape(x.shape), counts, psum.reshape(n_experts),
            jnp.array(0, dtype=jnp.int32))


# speedup vs baseline: 3.4497x; 1.0819x over previous
import functools

import jax
import jax.numpy as jnp
from jax import lax
from jax.experimental import pallas as pl
from jax.experimental.pallas import tpu as pltpu
from jax.experimental.pallas import tpu_sc as plsc

_NC, _NS = 2, 16
_NW = _NC * _NS
_BLK = 128


def _router_body(n_experts, x_ref, Wsw_ref, bsw_ref, xs_ref, routes_ref,
                 rl_ref, hist_ref, psum_ref):
    x = x_ref[...]
    logits = jnp.dot(x, Wsw_ref[...],
                     preferred_element_type=jnp.float32) + bsw_ref[...]
    m = jnp.max(logits, axis=-1, keepdims=True)
    ex = jnp.exp(logits - m)
    p = ex / jnp.sum(ex, axis=-1, keepdims=True)
    pmax = jnp.max(p, axis=-1, keepdims=True)
    iota_e = lax.broadcasted_iota(jnp.int32, p.shape, 1)
    routes = jnp.min(jnp.where(p >= pmax, iota_e, n_experts), axis=-1,
                     keepdims=True)
    xs_ref[...] = x * pmax
    routes_ref[...] = routes
    onehot = (routes == iota_e).astype(jnp.float32)
    tb = x.shape[0]
    ri = lax.broadcasted_iota(jnp.int32, (tb, tb), 0)
    ci = lax.broadcasted_iota(jnp.int32, (tb, tb), 1)
    tri = (ri > ci).astype(jnp.float32)
    before = jnp.dot(tri, onehot, preferred_element_type=jnp.float32)
    rl = jnp.sum(before * onehot, axis=-1, keepdims=True)
    rl_ref[...] = rl.astype(jnp.int32)
    hist_ref[...] = jnp.sum(onehot, axis=0).reshape(1, 1, n_experts)

    @pl.when(pl.program_id(0) == 0)
    def _():
        psum_ref[...] = jnp.zeros_like(psum_ref)

    psum_ref[...] += jnp.sum(p, axis=0, keepdims=True)


def _dispatch_body(xs_hbm, routes_hbm, rl_hbm, tflat_hbm, xsorted_hbm,
                   pos_hbm, routes_v, rl_v, t_v, pmm, rows0, rows1,
                   sg0, sg1, ss0, ss1):
    w = lax.axis_index("s") * _NC + lax.axis_index("c")
    base = w * 256
    pltpu.sync_copy(routes_hbm.at[pl.ds(base, 256)], routes_v)
    pltpu.sync_copy(rl_hbm.at[pl.ds(base, 256)], rl_v)
    pltpu.sync_copy(tflat_hbm.at[pl.ds(w * 8, 8)], t_v)
    for j in range(8):
        for g in range(2):
            o = j * 32 + g * 16
            rv = routes_v[pl.ds(o, 16)]
            bv = plsc.load_gather(t_v, [rv])
            pmm[j, pl.ds(g * 16, 16)] = bv + rl_v[pl.ds(o, 16)]
    for j in range(8):
        pltpu.sync_copy(pmm.at[j], pos_hbm.at[w * 8 + j])
    rows = (rows0, rows1)
    sg = (sg0, sg1)
    ss = (ss0, ss1)
    pend_g = {}
    pend_s = {}
    pend_g[0] = pltpu.async_copy(xs_hbm.at[pl.ds(base, 32)], rows[0], sg[0])
    for j in range(8):
        b = j % 2
        pend_g[j].wait()
        pend_s[j] = pltpu.async_copy(rows[b], xsorted_hbm.at[pmm.at[j]],
                                     ss[b])
        if j + 1 < 8:
            nb = (j + 1) % 2
            if j >= 1:
                pend_s[j - 1].wait()
            pend_g[j + 1] = pltpu.async_copy(
                xs_hbm.at[pl.ds(base + (j + 1) * 32, 32)], rows[nb], sg[nb])
    pend_s[6].wait()
    pend_s[7].wait()


def _wconv_body(W1_ref, W1b_ref):
    W1b_ref[...] = W1_ref[...].astype(jnp.bfloat16)


def _ffn_body(be_ref, x_ref, W1_ref, b1_ref, W2_ref, b2_ref, out_ref):
    xb = x_ref[...].astype(jnp.bfloat16)
    h = jnp.maximum(
        jnp.dot(xb, W1_ref[0], preferred_element_type=jnp.float32)
        + b1_ref[0], 0.0)
    y = jnp.dot(h, W2_ref[0],
                preferred_element_type=jnp.float32) + b2_ref[0]
    out_ref[...] = y


def _combine_body(ysorted_hbm, pos_hbm, final_hbm, pmm, rows0, rows1,
                  sg0, sg1, ss0, ss1):
    w = lax.axis_index("s") * _NC + lax.axis_index("c")
    base = w * 256
    for j in range(8):
        pltpu.sync_copy(pos_hbm.at[w * 8 + j], pmm.at[j])
    rows = (rows0, rows1)
    sg = (sg0, sg1)
    ss = (ss0, ss1)
    pend_g = {}
    pend_s = {}
    pend_g[0] = pltpu.async_copy(ysorted_hbm.at[pmm.at[0]], rows[0], sg[0])
    for j in range(8):
        b = j % 2
        pend_g[j].wait()
        pend_s[j] = pltpu.async_copy(rows[b],
                                     final_hbm.at[pl.ds(base + j * 32, 32)],
                                     ss[b])
        if j + 1 < 8:
            nb = (j + 1) % 2
            if j >= 1:
                pend_s[j - 1].wait()
            pend_g[j + 1] = pltpu.async_copy(ysorted_hbm.at[pmm.at[j + 1]],
                                             rows[nb], sg[nb])
    pend_s[6].wait()
    pend_s[7].wait()


def kernel(x, W_switch, b_switch, W1, b1, W2, b2):
    seq_len, batch, d_model = x.shape
    n_experts, _, d_ff = W1.shape
    n = seq_len * batch
    tb = 1024
    ntb = n // tb
    xf = x.reshape(n, d_model)

    xs, routes2, rl2, hist3, psum = pl.pallas_call(
        functools.partial(_router_body, n_experts),
        grid=(ntb,),
        in_specs=[
            pl.BlockSpec((tb, d_model), lambda t: (t, 0)),
            pl.BlockSpec((d_model, n_experts), lambda t: (0, 0)),
            pl.BlockSpec((1, n_experts), lambda t: (0, 0)),
        ],
        out_specs=[
            pl.BlockSpec((tb, d_model), lambda t: (t, 0)),
            pl.BlockSpec((tb, 1), lambda t: (t, 0)),
            pl.BlockSpec((tb, 1), lambda t: (t, 0)),
            pl.BlockSpec((1, 1, n_experts), lambda t: (t, 0, 0)),
            pl.BlockSpec((1, n_experts), lambda t: (0, 0)),
        ],
        out_shape=[
            jax.ShapeDtypeStruct((n, d_model), jnp.float32),
            jax.ShapeDtypeStruct((n, 1), jnp.int32),
            jax.ShapeDtypeStruct((n, 1), jnp.int32),
            jax.ShapeDtypeStruct((ntb, 1, n_experts), jnp.float32),
            jax.ShapeDtypeStruct((1, n_experts), jnp.float32),
        ],
    )(xf, W_switch, b_switch.reshape(1, n_experts))

    hist = hist3.reshape(ntb, n_experts)
    counts = jnp.sum(hist, axis=0)
    counts_i = counts.astype(jnp.int32)
    block_base = (jnp.cumsum(hist, axis=0) - hist).astype(jnp.int32)
    sizes = ((counts_i + (_BLK - 1)) // _BLK) * _BLK
    ends = jnp.cumsum(sizes)
    starts = ends - sizes
    p_tot = n + n_experts * _BLK
    nblk = p_tot // _BLK
    blk_lo = jnp.arange(nblk, dtype=jnp.int32) * _BLK
    block_expert = jnp.minimum(
        jnp.sum((blk_lo[:, None] >= ends[None, :]).astype(jnp.int32), axis=1),
        n_experts - 1).astype(jnp.int32)
    t_tab = starts[None, :] + jnp.repeat(block_base, tb // (n // _NW), axis=0)
    tflat = t_tab.reshape(-1).astype(jnp.int32)

    mesh = plsc.VectorSubcoreMesh(core_axis_name="c", subcore_axis_name="s")
    x_sorted, pos2d = pl.kernel(
        _dispatch_body,
        out_type=[
            jax.ShapeDtypeStruct((p_tot, d_model), jnp.float32),
            jax.ShapeDtypeStruct((n // 32, 32), jnp.int32),
        ],
        mesh=mesh,
        compiler_params=pltpu.CompilerParams(needs_layout_passes=False),
        scratch_types=[
            pltpu.VMEM((256,), jnp.int32),
            pltpu.VMEM((256,), jnp.int32),
            pltpu.VMEM((8,), jnp.int32),
            pltpu.VMEM((8, 32), jnp.int32),
            pltpu.VMEM((32, d_model), jnp.float32),
            pltpu.VMEM((32, d_model), jnp.float32),
            pltpu.SemaphoreType.DMA,
            pltpu.SemaphoreType.DMA,
            pltpu.SemaphoreType.DMA,
            pltpu.SemaphoreType.DMA,
        ],
    )(xs, routes2.reshape(n), rl2.reshape(n), tflat)

    W1b = pl.pallas_call(
        _wconv_body,
        grid=(n_experts, 2),
        in_specs=[
            pl.BlockSpec((1, d_model, d_ff // 2), lambda e, c: (e, 0, c)),
        ],
        out_specs=pl.BlockSpec((1, d_model, d_ff // 2), lambda e, c: (e, 0, c)),
        out_shape=jax.ShapeDtypeStruct((n_experts, d_model, d_ff),
                                       jnp.bfloat16),
    )(W1)

    grid_spec = pltpu.PrefetchScalarGridSpec(
        num_scalar_prefetch=1,
        grid=(nblk,),
        in_specs=[
            pl.BlockSpec((_BLK, d_model), lambda t, be: (t, 0)),
            pl.BlockSpec((1, d_model, d_ff), lambda t, be: (be[t], 0, 0)),
            pl.BlockSpec((1, 1, d_ff), lambda t, be: (be[t], 0, 0)),
            pl.BlockSpec((1, d_ff, d_model), lambda t, be: (be[t], 0, 0)),
            pl.BlockSpec((1, 1, d_model), lambda t, be: (be[t], 0, 0)),
        ],
        out_specs=pl.BlockSpec((_BLK, d_model), lambda t, be: (t, 0)),
    )
    y_sorted = pl.pallas_call(
        _ffn_body,
        grid_spec=grid_spec,
        out_shape=jax.ShapeDtypeStruct((p_tot, d_model), jnp.float32),
    )(block_expert, x_sorted, W1b,
      b1.reshape(n_experts, 1, d_ff), W2,
      b2.reshape(n_experts, 1, d_model))

    final = pl.kernel(
        _combine_body,
        out_type=jax.ShapeDtypeStruct((n, d_model), jnp.float32),
        mesh=mesh,
        scratch_types=[
            pltpu.VMEM((8, 32), jnp.int32),
            pltpu.VMEM((32, d_model), jnp.float32),
            pltpu.VMEM((32, d_model), jnp.float32),
            pltpu.SemaphoreType.DMA,
            pltpu.SemaphoreType.DMA,
            pltpu.SemaphoreType.DMA,
            pltpu.SemaphoreType.DMA,
        ],
    )(y_sorted, pos2d)

    return (final.reshape(x.shape), counts, psum.reshape(n_experts),
            jnp.array(0, dtype=jnp.int32))
